# Initial kernel scaffold; baseline (speedup 1.0000x reference)
import jax
import jax.numpy as jnp
from jax.experimental import pallas as pl


def kernel(x, edge_index, batch_size, framework, W1l, b1l, W1r, W2l, b2l, W2r, W3l, b3l, W3r):
    raise NotImplementedError("write your pallas kernel here")


# speedup vs baseline: 6.3234x; 6.3234x over previous
import functools

import jax
import jax.numpy as jnp
from jax import lax
from jax.experimental import pallas as pl
from jax.experimental.pallas import tpu as pltpu
from jax.experimental.pallas import tpu_sc as plsc

N = 10000
E = 320000
D = 128
NT = 16
ROWS_PT = 624
TAIL0 = ROWS_PT * NT
TAILN = N - TAIL0
A = 0.25
BLK = 200

_MESH = plsc.VectorSubcoreMesh(core_axis_name="c", subcore_axis_name="s")


def _acc_init(zeros_hbm, acc, s):
    row0 = s * ROWS_PT
    pltpu.sync_copy(zeros_hbm.at[pl.ds(row0, ROWS_PT)],
                    acc.at[pl.ds(row0, ROWS_PT)])

    @pl.when(s == NT - 1)
    def _():
        pltpu.sync_copy(zeros_hbm.at[pl.ds(TAIL0, TAILN)],
                        acc.at[pl.ds(TAIL0, TAILN)])


def _acc_writeback(acc, out_hbm, c, s):
    row0 = s * ROWS_PT
    pltpu.sync_copy(acc.at[pl.ds(row0, ROWS_PT)],
                    out_hbm.at[pl.ds(c * N + row0, ROWS_PT)])

    @pl.when(s == NT - 1)
    def _():
        pltpu.sync_copy(acc.at[pl.ds(TAIL0, TAILN)],
                        out_hbm.at[pl.ds(c * N + TAIL0, TAILN)])


def _make_counts():
    ept = (E // 2) // NT

    @functools.partial(
        pl.kernel,
        out_type=jax.ShapeDtypeStruct((2 * N, D), jnp.float32),
        mesh=_MESH,
        scratch_types=[
            pltpu.VMEM((BLK,), jnp.int32),
            pltpu.VMEM((BLK, D), jnp.float32),
            pltpu.VMEM_SHARED((N, D), jnp.float32),
        ],
    )
    def counts_kernel(dst_hbm, ones_hbm, zeros_hbm, out_hbm, idx_v, ones_v, acc):
        c = lax.axis_index("c")
        s = lax.axis_index("s")
        pltpu.sync_copy(ones_hbm, ones_v)
        _acc_init(zeros_hbm, acc, s)
        plsc.subcore_barrier()

        base = c * (E // 2) + s * ept

        def body(k, _):
            pltpu.sync_copy(dst_hbm.at[pl.ds(base + k * BLK, BLK)], idx_v)
            pltpu.sync_copy(ones_v, acc.at[idx_v], add=True)
            return 0
        lax.fori_loop(0, ept // BLK, body, 0)
        plsc.subcore_barrier()
        _acc_writeback(acc, out_hbm, c, s)

    return counts_kernel


def _make_segsum_es():
    ept = (E // 2) // NT

    @functools.partial(
        pl.kernel,
        out_type=jax.ShapeDtypeStruct((2 * N, D), jnp.float32),
        mesh=_MESH,
        scratch_types=[
            pltpu.VMEM((BLK,), jnp.int32),
            pltpu.VMEM((BLK,), jnp.int32),
            pltpu.VMEM((BLK, D), jnp.float32),
            pltpu.VMEM_SHARED((N, D), jnp.float32),
            pltpu.SemaphoreType.DMA,
        ],
    )
    def segsum_kernel(table_hbm, src_hbm, dst_hbm, zeros_hbm, out_hbm,
                      sidx_v, didx_v, rows_v, acc, sem):
        c = lax.axis_index("c")
        s = lax.axis_index("s")
        _acc_init(zeros_hbm, acc, s)
        plsc.subcore_barrier()

        base = c * (E // 2) + s * ept

        def body(k, _):
            pltpu.sync_copy(src_hbm.at[pl.ds(base + k * BLK, BLK)], sidx_v)
            pltpu.async_copy(table_hbm.at[sidx_v], rows_v, sem).wait()
            pltpu.sync_copy(dst_hbm.at[pl.ds(base + k * BLK, BLK)], didx_v)
            pltpu.sync_copy(rows_v, acc.at[didx_v], add=True)
            return 0
        lax.fori_loop(0, ept // BLK, body, 0)
        plsc.subcore_barrier()
        _acc_writeback(acc, out_hbm, c, s)

    return segsum_kernel


def _make_segsum_fs():
    ept = E // NT

    @functools.partial(
        pl.kernel,
        out_type=jax.ShapeDtypeStruct((2 * N, D), jnp.float32),
        mesh=_MESH,
        scratch_types=[
            pltpu.VMEM((BLK,), jnp.int32),
            pltpu.VMEM((BLK,), jnp.int32),
            pltpu.VMEM((BLK, D), jnp.float32),
            pltpu.VMEM_SHARED((N, D), jnp.float32),
            pltpu.SemaphoreType.DMA,
        ],
    )
    def segsum_kernel(table_hbm, src2_hbm, dst_hbm, zeros_hbm, out_hbm,
                      sidx_v, didx_v, rows_v, acc, sem):
        c = lax.axis_index("c")
        s = lax.axis_index("s")
        _acc_init(zeros_hbm, acc, s)
        plsc.subcore_barrier()

        sbase = c * E + s * ept
        dbase = s * ept

        def body(k, _):
            pltpu.sync_copy(src2_hbm.at[pl.ds(sbase + k * BLK, BLK)], sidx_v)
            pltpu.async_copy(table_hbm.at[sidx_v], rows_v, sem).wait()
            pltpu.sync_copy(dst_hbm.at[pl.ds(dbase + k * BLK, BLK)], didx_v)
            pltpu.sync_copy(rows_v, acc.at[didx_v], add=True)
            return 0
        lax.fori_loop(0, ept // BLK, body, 0)
        plsc.subcore_barrier()
        _acc_writeback(acc, out_hbm, c, s)

    return segsum_kernel


_counts_sc = _make_counts()
_segsum_es = _make_segsum_es()
_segsum_fs = _make_segsum_fs()


R = 1000
_GRID = (N // R,)
_TC_PARAMS = pltpu.CompilerParams(dimension_semantics=("arbitrary",))


def _recip_cnt(cnt_ref):
    tot = cnt_ref[0][:, :1] + cnt_ref[1][:, :1]
    return 1.0 / jnp.maximum(tot, 1.0)


def _prelu_tc(h):
    return jnp.where(h >= 0, h, A * h)


def _tc1_body(s1_ref, cnt_ref, x_ref, wl_ref, b_ref, wr_ref, out_ref):
    mean = (s1_ref[0] + s1_ref[1]) * _recip_cnt(cnt_ref)
    h = (jnp.dot(mean, wl_ref[...], preferred_element_type=jnp.float32)
         + b_ref[...]
         + jnp.dot(x_ref[...], wr_ref[...], preferred_element_type=jnp.float32))
    h = _prelu_tc(h)
    out_ref[0] = h[:, :128]
    out_ref[1] = h[:, 128:]


def _tc2_body(s2_ref, cnt_ref, h1_ref, w2l_ref, b_ref, w2r_ref, w3l_ref,
              h2_ref, p3_ref):
    mean = jnp.concatenate([s2_ref[0], s2_ref[1]], axis=1) * _recip_cnt(cnt_ref)
    h1 = jnp.concatenate([h1_ref[0], h1_ref[1]], axis=1)
    h2 = (jnp.dot(mean, w2l_ref[...], preferred_element_type=jnp.float32)
          + b_ref[...]
          + jnp.dot(h1, w2r_ref[...], preferred_element_type=jnp.float32))
    h2 = _prelu_tc(h2)
    p3 = jnp.dot(h2, w3l_ref[...], preferred_element_type=jnp.float32)
    h2_ref[0] = h2[:, :128]
    h2_ref[1] = h2[:, 128:]
    p3_ref[...] = p3


def _tc3_body(s3_ref, cnt_ref, h2_ref, b_ref, w3r_ref, bs_ref, fw_ref,
              out_ref):
    mean3 = (s3_ref[0] + s3_ref[1]) * _recip_cnt(cnt_ref)
    h2 = jnp.concatenate([h2_ref[0], h2_ref[1]], axis=1)
    h3 = (mean3 + b_ref[...]
          + jnp.dot(h2, w3r_ref[...], preferred_element_type=jnp.float32))
    h3 = _prelu_tc(h3)
    rows = pl.program_id(0) * R + lax.broadcasted_iota(jnp.int32, (R, 128), 0)
    keep = (fw_ref[0] != 0) | (rows < bs_ref[0])
    out_ref[...] = jnp.where(keep, h3, 0.0)


def _blk2(dh):
    return pl.BlockSpec((2, R, dh), lambda i: (0, i, 0))


def _blk(dh):
    return pl.BlockSpec((R, dh), lambda i: (i, 0))


def _wspec(k, n):
    return pl.BlockSpec((k, n), lambda i: (0, 0))


_SMEM1 = pl.BlockSpec(memory_space=pltpu.SMEM)


def _tc1(s1, cnt, x, wl, b, wr):
    return pl.pallas_call(
        _tc1_body,
        grid=_GRID,
        in_specs=[_blk2(128), _blk2(128), _blk(128),
                  _wspec(128, 256), _wspec(1, 256), _wspec(128, 256)],
        out_specs=_blk2(128),
        out_shape=jax.ShapeDtypeStruct((2, N, 128), jnp.float32),
        compiler_params=_TC_PARAMS,
    )(s1, cnt, x, wl, b, wr)


def _tc2(s2, cnt, h1, w2l, b, w2r, w3l):
    return pl.pallas_call(
        _tc2_body,
        grid=_GRID,
        in_specs=[_blk2(128), _blk2(128), _blk2(128),
                  _wspec(256, 256), _wspec(1, 256), _wspec(256, 256),
                  _wspec(256, 128)],
        out_specs=[_blk2(128), _blk(128)],
        out_shape=[jax.ShapeDtypeStruct((2, N, 128), jnp.float32),
                   jax.ShapeDtypeStruct((N, 128), jnp.float32)],
        compiler_params=_TC_PARAMS,
    )(s2, cnt, h1, w2l, b, w2r, w3l)


def _tc3(s3, cnt, h2, b, w3r, bs, fw):
    return pl.pallas_call(
        _tc3_body,
        grid=_GRID,
        in_specs=[_blk2(128), _blk2(128), _blk2(128),
                  _wspec(1, 128), _wspec(256, 128), _SMEM1, _SMEM1],
        out_specs=pl.BlockSpec((R, 128), lambda i: (i, 0)),
        out_shape=jax.ShapeDtypeStruct((N, 128), jnp.float32),
        compiler_params=_TC_PARAMS,
    )(s3, cnt, h2, b, w3r, bs, fw)


def kernel(x, edge_index, batch_size, framework,
           W1l, b1l, W1r, W2l, b2l, W2r, W3l, b3l, W3r):
    src = edge_index[0]
    dst = edge_index[1]
    src2 = jnp.concatenate([src, src + N])

    z128 = jnp.zeros((N, 128), jnp.float32)
    bs = jnp.asarray(batch_size, jnp.int32).reshape(1)
    fw = jnp.asarray(framework, jnp.int32).reshape(1)

    ones128 = jnp.ones((BLK, 128), jnp.float32)
    cnt = _counts_sc(dst, ones128, z128).reshape(2, N, 128)
    s1 = _segsum_es(x, src, dst, z128).reshape(2, N, 128)
    h1 = _tc1(s1, cnt, x, W1l.T, b1l.reshape(1, 256), W1r.T)

    h1f = h1.reshape(2 * N, 128)
    s2 = _segsum_fs(h1f, src2, dst, z128).reshape(2, N, 128)
    h2, p3 = _tc2(s2, cnt, h1, W2l.T, b2l.reshape(1, 256), W2r.T, W3l.T)

    s3 = _segsum_es(p3, src, dst, z128).reshape(2, N, 128)
    out = _tc3(s3, cnt, h2, b3l.reshape(1, 128), W3r.T, bs, fw)
    return out
